# Initial kernel scaffold; baseline (speedup 1.0000x reference)
import jax
import jax.numpy as jnp
from jax.experimental import pallas as pl


def kernel(x, edge_index, edge_attr, W_e, b_e, W1, b1, gamma, beta, W2, b2):
    raise NotImplementedError("write your pallas kernel here")


# speedup vs baseline: 2.5619x; 2.5619x over previous
import functools

import jax
import jax.numpy as jnp
from jax import lax
from jax.experimental import pallas as pl
from jax.experimental.pallas import tpu as pltpu
from jax.experimental.pallas import tpu_sc as plsc

N_NODES = 10000
N_EDGES = 320000
D = 128
D_EDGE = 16

C = 128
NSUB = C // 128
NUM_CHUNKS = N_EDGES // C
NC = 2
NS = 16
NW = NC * NS
T = (NUM_CHUNKS + NW - 1) // NW
ROWS_A = 624
ROWS_LAST = N_NODES - (NS - 1) * ROWS_A


def _edge_linear(edge_attr, W_e, b_e):
    EB = 2000

    def body(a_ref, w_ref, b_ref, o_ref):
        o_ref[...] = lax.dot_general(
            a_ref[...], w_ref[...], (((1,), (1,)), ((), ())),
            preferred_element_type=jnp.float32) + b_ref[...]

    return pl.pallas_call(
        body,
        grid=(N_EDGES // EB,),
        in_specs=[
            pl.BlockSpec((EB, D_EDGE), lambda i: (i, 0)),
            pl.BlockSpec((D, D_EDGE), lambda i: (0, 0)),
            pl.BlockSpec((1, D), lambda i: (0, 0)),
        ],
        out_specs=pl.BlockSpec((EB, D), lambda i: (i, 0)),
        out_shape=jax.ShapeDtypeStruct((N_EDGES, D), jnp.float32),
    )(edge_attr, W_e, b_e.reshape(1, D))


def _sc_aggregate(x, e, src2d, dst2d, zeros):
    mesh = plsc.VectorSubcoreMesh(core_axis_name="c", subcore_axis_name="s")

    @functools.partial(
        pl.kernel,
        out_type=jax.ShapeDtypeStruct((NC, N_NODES, D), jnp.float32),
        mesh=mesh,
        scratch_types=[
            pltpu.VMEM((NSUB, 128), jnp.int32),
            pltpu.VMEM((NSUB, 128), jnp.int32),
            pltpu.VMEM((C, D), jnp.float32),
            pltpu.VMEM((C, D), jnp.float32),
            pltpu.VMEM_SHARED((N_NODES, D), jnp.float32),
            pltpu.SemaphoreType.DMA,
        ],
    )
    def body(x_hbm, e_hbm, src_hbm, dst_hbm, z_hbm, out_hbm,
             idx_s, idx_d, xrows, erows, shared, sem):
        cid = lax.axis_index("c")
        sid = lax.axis_index("s")
        wid = sid * NC + cid

        @pl.when(sid < NS - 1)
        def _():
            pltpu.sync_copy(z_hbm.at[pl.ds(0, ROWS_A)],
                            shared.at[pl.ds(sid * ROWS_A, ROWS_A)])

        @pl.when(sid == NS - 1)
        def _():
            pltpu.sync_copy(z_hbm.at[pl.ds(0, ROWS_LAST)],
                            shared.at[pl.ds((NS - 1) * ROWS_A, ROWS_LAST)])

        plsc.subcore_barrier()

        def chunk_body(t, carry):
            c = t * NW + wid

            @pl.when(c < NUM_CHUNKS)
            def _():
                pltpu.sync_copy(src_hbm.at[pl.ds(c * NSUB, NSUB)], idx_s)
                pltpu.sync_copy(dst_hbm.at[pl.ds(c * NSUB, NSUB)], idx_d)
                pltpu.sync_copy(e_hbm.at[pl.ds(c * C, C)], erows)
                cps = [
                    pltpu.async_copy(x_hbm.at[idx_s.at[j]],
                                     xrows.at[pl.ds(j * 128, 128)], sem)
                    for j in range(NSUB)
                ]
                for cp in cps:
                    cp.wait()

                def row_body(r, _):
                    for j in range(D // 16):
                        v = xrows[r, pl.ds(j * 16, 16)] + erows[r, pl.ds(j * 16, 16)]
                        erows[r, pl.ds(j * 16, 16)] = jnp.maximum(v, 0.0)
                    return 0

                lax.fori_loop(0, C, row_body, 0)

                for j in range(NSUB):
                    pltpu.sync_copy(erows.at[pl.ds(j * 128, 128)],
                                    shared.at[idx_d.at[j]], add=True)

            return carry

        lax.fori_loop(0, T, chunk_body, 0)

        plsc.subcore_barrier()

        @pl.when(sid < NS - 1)
        def _():
            pltpu.sync_copy(shared.at[pl.ds(sid * ROWS_A, ROWS_A)],
                            out_hbm.at[cid, pl.ds(sid * ROWS_A, ROWS_A)])

        @pl.when(sid == NS - 1)
        def _():
            pltpu.sync_copy(shared.at[pl.ds((NS - 1) * ROWS_A, ROWS_LAST)],
                            out_hbm.at[cid, pl.ds((NS - 1) * ROWS_A, ROWS_LAST)])

    return body(x, e, src2d, dst2d, zeros)


def _mlp(x, p, W1, b1, gamma, beta, W2, b2):
    def body(x_ref, p_ref, w1_ref, b1_ref, g_ref, be_ref, w2_ref, b2_ref, o_ref):
        h = x_ref[...] + p_ref[0] + p_ref[1]
        h = lax.dot_general(h, w1_ref[...], (((1,), (1,)), ((), ())),
                            preferred_element_type=jnp.float32) + b1_ref[...]
        mu = jnp.mean(h, axis=0, keepdims=True)
        var = jnp.mean((h - mu) ** 2, axis=0, keepdims=True)
        h = (h - mu) / jnp.sqrt(var + 1e-5) * g_ref[...] + be_ref[...]
        h = jnp.maximum(h, 0.0)
        o_ref[...] = lax.dot_general(h, w2_ref[...], (((1,), (1,)), ((), ())),
                                     preferred_element_type=jnp.float32) + b2_ref[...]

    return pl.pallas_call(
        body,
        out_shape=jax.ShapeDtypeStruct((N_NODES, D), jnp.float32),
    )(x, p, W1, b1.reshape(1, D), gamma.reshape(1, D), beta.reshape(1, D),
      W2, b2.reshape(1, D))


def kernel(x, edge_index, edge_attr, W_e, b_e, W1, b1, gamma, beta, W2, b2):
    ei = edge_index.astype(jnp.int32)
    src2d = ei[0].reshape(NUM_CHUNKS * NSUB, 128)
    dst2d = ei[1].reshape(NUM_CHUNKS * NSUB, 128)
    e = _edge_linear(edge_attr, W_e, b_e)
    zeros = jnp.zeros((ROWS_LAST, D), jnp.float32)
    p = _sc_aggregate(x, e, src2d, dst2d, zeros)
    return _mlp(x, p, W1, b1, gamma, beta, W2, b2)
